# Initial kernel scaffold; baseline (speedup 1.0000x reference)
import jax
import jax.numpy as jnp
from jax.experimental import pallas as pl


def kernel(x, support, W1, b1, gamma1, beta1, W2, b2, gamma2, beta2):
    raise NotImplementedError("write your pallas kernel here")


# speedup vs baseline: 1.0132x; 1.0132x over previous
import jax
import jax.numpy as jnp
from jax.experimental import pallas as pl

_EPS = 1e-5


def _xw_kernel(x_ref, w_ref, out_ref):
    out_ref[...] = jnp.dot(
        x_ref[...].astype(jnp.bfloat16),
        w_ref[...].astype(jnp.bfloat16),
        preferred_element_type=jnp.float32,
    ).astype(jnp.bfloat16)


def _spmm_kernel(sup_ref, h_ref, b_ref, z_ref, stats_ref):
    a = jnp.dot(
        sup_ref[...].astype(jnp.bfloat16),
        h_ref[...],
        preferred_element_type=jnp.float32,
    )
    z = jnp.maximum(a + b_ref[...], 0.0)
    z_ref[...] = z.astype(jnp.bfloat16)
    st = jnp.concatenate(
        [jnp.sum(z, axis=0, keepdims=True),
         jnp.sum(z * z, axis=0, keepdims=True)],
        axis=0,
    )

    @pl.when(pl.program_id(0) == 0)
    def _():
        stats_ref[...] = st

    @pl.when(pl.program_id(0) != 0)
    def _():
        stats_ref[...] += st


def _bn_affine(stats_ref, gamma_ref, beta_ref, n_rows):
    mu = stats_ref[0:1, :] / n_rows
    var = stats_ref[1:2, :] / n_rows - mu * mu
    s = gamma_ref[...] * jax.lax.rsqrt(var + _EPS)
    t = beta_ref[...] - mu * s
    return s, t


def _proj_kernel(z_ref, stats_ref, gamma_ref, beta_ref, w_ref, g_ref):
    n_rows = z_ref.shape[0]
    s, t = _bn_affine(stats_ref, gamma_ref, beta_ref, n_rows)
    h = z_ref[...].astype(jnp.float32) * s + t
    g_ref[...] = jnp.dot(
        h.astype(jnp.bfloat16),
        w_ref[...].astype(jnp.bfloat16),
        preferred_element_type=jnp.float32,
    ).astype(jnp.bfloat16)


def _bn_kernel(y_ref, stats_ref, gamma_ref, beta_ref, out_ref):
    n_rows = y_ref.shape[0]
    s, t = _bn_affine(stats_ref, gamma_ref, beta_ref, n_rows)
    out_ref[...] = y_ref[...].astype(jnp.float32) * s + t


def _spmm_fused(support, h_b16, bias, block_m):
    n = support.shape[0]
    d = h_b16.shape[1]
    grid = n // block_m
    z, stats = pl.pallas_call(
        _spmm_kernel,
        grid=(grid,),
        in_specs=[
            pl.BlockSpec((block_m, n), lambda i: (i, 0)),
            pl.BlockSpec((n, d), lambda i: (0, 0)),
            pl.BlockSpec((1, d), lambda i: (0, 0)),
        ],
        out_specs=[
            pl.BlockSpec((block_m, d), lambda i: (i, 0)),
            pl.BlockSpec((2, d), lambda i: (0, 0)),
        ],
        out_shape=[
            jax.ShapeDtypeStruct((n, d), jnp.bfloat16),
            jax.ShapeDtypeStruct((2, d), jnp.float32),
        ],
    )(support, h_b16, bias)
    return z, stats


def kernel(x, support, W1, b1, gamma1, beta1, W2, b2, gamma2, beta2):
    n, d_in = x.shape
    d_h = W1.shape[1]
    d_out = W2.shape[1]
    block_m = 400 if n % 400 == 0 else (200 if n % 200 == 0 else n)

    b1r = b1.reshape(1, d_h)
    g1r = gamma1.reshape(1, d_h)
    be1r = beta1.reshape(1, d_h)
    b2r = b2.reshape(1, d_out)
    g2r = gamma2.reshape(1, d_out)
    be2r = beta2.reshape(1, d_out)

    h0 = pl.pallas_call(
        _xw_kernel,
        out_shape=jax.ShapeDtypeStruct((n, d_h), jnp.bfloat16),
    )(x, W1)

    z, stats1 = _spmm_fused(support, h0, b1r, block_m)

    g = pl.pallas_call(
        _proj_kernel,
        out_shape=jax.ShapeDtypeStruct((n, d_out), jnp.bfloat16),
    )(z, stats1, g1r, be1r, W2)

    y, stats2 = _spmm_fused(support, g, b2r, block_m)

    out = pl.pallas_call(
        _bn_kernel,
        out_shape=jax.ShapeDtypeStruct((n, d_out), jnp.float32),
    )(y, stats2, g2r, be2r)

    return (out, support)


# speedup vs baseline: 1.0143x; 1.0011x over previous
import jax
import jax.numpy as jnp
from jax.experimental import pallas as pl
from jax.experimental.pallas import tpu as pltpu

_EPS = 1e-5


def _bn_affine(stats, gamma, beta, n_rows):
    mu = stats[0:1, :] / n_rows
    var = stats[1:2, :] / n_rows - mu * mu
    s = gamma * jax.lax.rsqrt(var + _EPS)
    t = beta - mu * s
    return s, t


def _make_fused_kernel(n, p, block_m):
    def fused(sup_ref, x_ref, w1_ref, w2_ref, b1_ref, g1_ref, be1_ref,
              b2_ref, g2_ref, be2_ref, out_ref,
              h0_s, z_s, g_s, y_s, st1_s, st2_s):
        i = pl.program_id(0)

        @pl.when(i == 0)
        def _():
            h0_s[...] = jnp.dot(
                x_ref[...], w1_ref[...],
                preferred_element_type=jnp.float32,
            )

        @pl.when(i < p)
        def _():
            a = jnp.dot(
                sup_ref[...],
                h0_s[...],
                preferred_element_type=jnp.float32,
            )
            z = jnp.maximum(a + b1_ref[...], 0.0)
            z_s[pl.ds(i * block_m, block_m), :] = z
            st = jnp.concatenate(
                [jnp.sum(z, axis=0, keepdims=True),
                 jnp.sum(z * z, axis=0, keepdims=True)], axis=0)

            @pl.when(i == 0)
            def _():
                st1_s[...] = st

            @pl.when(i != 0)
            def _():
                st1_s[...] += st

        @pl.when(i == p)
        def _():
            s1, t1 = _bn_affine(st1_s[...], g1_ref[...], be1_ref[...], n)
            h = z_s[...] * s1 + t1
            g_s[...] = jnp.dot(
                h, w2_ref[...],
                preferred_element_type=jnp.float32,
            )

        @pl.when(jnp.logical_and(i >= p, i < 2 * p))
        def _():
            a = jnp.dot(
                sup_ref[...],
                g_s[...],
                preferred_element_type=jnp.float32,
            )
            y = jnp.maximum(a + b2_ref[...], 0.0)
            y_s[pl.ds((i - p) * block_m, block_m), :] = y
            st = jnp.concatenate(
                [jnp.sum(y, axis=0, keepdims=True),
                 jnp.sum(y * y, axis=0, keepdims=True)], axis=0)

            @pl.when(i == p)
            def _():
                st2_s[...] = st

            @pl.when(i != p)
            def _():
                st2_s[...] += st

        @pl.when(i >= 2 * p)
        def _():
            s2, t2 = _bn_affine(st2_s[...], g2_ref[...], be2_ref[...], n)
            yb = y_s[pl.ds((i - 2 * p) * block_m, block_m), :]
            out_ref[...] = yb * s2 + t2

    return fused


def kernel(x, support, W1, b1, gamma1, beta1, W2, b2, gamma2, beta2):
    n, d_in = x.shape
    d_h = W1.shape[1]
    d_out = W2.shape[1]
    block_m = next(bm for bm in (200, 100, n) if n % bm == 0)
    p = n // block_m

    def sup_idx(i):
        return (jnp.where(i < p, i, jnp.where(i < 2 * p, i - p, p - 1)), 0)

    def out_idx(i):
        return (jnp.where(i < 2 * p, 0, i - 2 * p), 0)

    const = lambda i: (0, 0)

    out = pl.pallas_call(
        _make_fused_kernel(n, p, block_m),
        grid=(3 * p,),
        in_specs=[
            pl.BlockSpec((block_m, n), sup_idx),
            pl.BlockSpec((n, d_in), const),
            pl.BlockSpec((d_in, d_h), const),
            pl.BlockSpec((d_h, d_out), const),
            pl.BlockSpec((1, d_h), const),
            pl.BlockSpec((1, d_h), const),
            pl.BlockSpec((1, d_h), const),
            pl.BlockSpec((1, d_out), const),
            pl.BlockSpec((1, d_out), const),
            pl.BlockSpec((1, d_out), const),
        ],
        out_specs=pl.BlockSpec((block_m, d_out), out_idx),
        out_shape=jax.ShapeDtypeStruct((n, d_out), jnp.float32),
        scratch_shapes=[
            pltpu.VMEM((n, d_h), jnp.float32),
            pltpu.VMEM((n, d_h), jnp.float32),
            pltpu.VMEM((n, d_out), jnp.float32),
            pltpu.VMEM((n, d_out), jnp.float32),
            pltpu.VMEM((2, d_h), jnp.float32),
            pltpu.VMEM((2, d_out), jnp.float32),
        ],
    )(support, x, W1, W2,
      b1.reshape(1, d_h), gamma1.reshape(1, d_h), beta1.reshape(1, d_h),
      b2.reshape(1, d_out), gamma2.reshape(1, d_out), beta2.reshape(1, d_out))

    return (out, support)
